# baseline (speedup 1.0000x reference)
import functools

import jax
import jax.numpy as jnp
from jax import lax
from jax.experimental import pallas as pl
from jax.experimental.pallas import tpu as pltpu
from jax.experimental.pallas import tpu_sc as plsc

N = 10000
E = 320000
D_IN = 200
D_OUT = 64

NC = 2
NS = 16
NW = NC * NS
EPW = E // NW
CHUNK = 80
NCHUNK = EPW // CHUNK
NPAD = 10240
RPT = NPAD // NS
RB = 80
ZCOPIES = RPT // RB
DP = 128
DEGD = 5

_mesh = plsc.VectorSubcoreMesh(core_axis_name="c", subcore_axis_name="s")


@functools.partial(
    pl.kernel,
    mesh=_mesh,
    out_type=jax.ShapeDtypeStruct((NC, NPAD), jnp.float32),
    scratch_types=[
        pltpu.VMEM((NCHUNK, CHUNK), jnp.int32),
        pltpu.VMEM((112,), jnp.float32),
        pltpu.VMEM((RPT,), jnp.float32),
        pltpu.VMEM_SHARED((NPAD,), jnp.float32),
        pltpu.SemaphoreType.DMA,
    ],
)
def _deg_kernel(dst_hbm, out_hbm, idx_v, ones_v, stage_v, acc_sh, sem):
    cid = lax.axis_index("c")
    sid = lax.axis_index("s")
    wid = cid * NS + sid

    pltpu.sync_copy(dst_hbm.at[wid], idx_v)

    for i in range(7):
        ones_v[pl.ds(i * 16, 16)] = jnp.ones((16,), jnp.float32)

    def _zero_body(i, _):
        stage_v[pl.ds(i * 16, 16)] = jnp.zeros((16,), jnp.float32)
        return _

    lax.fori_loop(0, RPT // 16, _zero_body, None)

    pltpu.sync_copy(stage_v, acc_sh.at[pl.ds(sid * RPT, RPT)])
    plsc.subcore_barrier()

    def _scatter_body(j, _):
        for k in range(DEGD):
            pltpu.async_copy(ones_v.at[pl.ds(0, CHUNK)],
                             acc_sh.at[idx_v.at[DEGD * j + k]], sem,
                             add=True)
        for k in range(DEGD):
            pltpu.make_async_copy(ones_v.at[pl.ds(0, CHUNK)],
                                  acc_sh.at[idx_v.at[DEGD * j + k]],
                                  sem).wait()
        return _

    lax.fori_loop(0, NCHUNK // DEGD, _scatter_body, None)
    plsc.subcore_barrier()

    pltpu.sync_copy(acc_sh.at[pl.ds(sid * RPT, RPT)], stage_v)
    pltpu.sync_copy(stage_v, out_hbm.at[cid, pl.ds(sid * RPT, RPT)])


@functools.partial(
    pl.kernel,
    mesh=_mesh,
    out_type=jax.ShapeDtypeStruct((NC, NPAD, DP), jnp.float32),
    scratch_types=[
        pltpu.VMEM((EPW,), jnp.int32),
        pltpu.VMEM((NCHUNK, CHUNK), jnp.int32),
        pltpu.VMEM((CHUNK, DP), jnp.float32),
        pltpu.VMEM((CHUNK, DP), jnp.float32),
        pltpu.VMEM_SHARED((NPAD, DP), jnp.float32),
        pltpu.SemaphoreType.DMA,
        pltpu.SemaphoreType.DMA,
        pltpu.SemaphoreType.DMA,
        pltpu.SemaphoreType.DMA,
    ],
)
def _agg_kernel(hp_hbm, src_hbm, dst_hbm, out_hbm,
                src_v, dst_v, rows_a, rows_b, acc_sh,
                sem_a, sem_b, sem_sa, sem_sb):
    cid = lax.axis_index("c")
    sid = lax.axis_index("s")
    wid = cid * NS + sid

    pltpu.sync_copy(src_hbm.at[wid], src_v)
    pltpu.sync_copy(dst_hbm.at[wid], dst_v)

    def _zero_body(i, _):
        for k in range(DP // 16):
            rows_a[i, pl.ds(k * 16, 16)] = jnp.zeros((16,), jnp.float32)
        return _

    lax.fori_loop(0, RB, _zero_body, None)
    for k in range(ZCOPIES):
        pltpu.sync_copy(rows_a.at[pl.ds(0, RB)],
                        acc_sh.at[pl.ds(sid * RPT + k * RB, RB)])
    plsc.subcore_barrier()

    def _sidx(j):
        return src_v.at[pl.ds(j * CHUNK, CHUNK)]

    pltpu.async_copy(hp_hbm.at[_sidx(0)], rows_a, sem_a)
    pltpu.async_copy(hp_hbm.at[_sidx(1)], rows_b, sem_b)

    def _edge_body(t, _):
        ja = 2 * t
        pltpu.make_async_copy(hp_hbm.at[_sidx(ja)], rows_a, sem_a).wait()
        pltpu.async_copy(rows_a, acc_sh.at[dst_v.at[ja]], sem_sa, add=True)
        pltpu.make_async_copy(hp_hbm.at[_sidx(ja + 1)], rows_b,
                              sem_b).wait()
        pltpu.async_copy(rows_b, acc_sh.at[dst_v.at[ja + 1]], sem_sb,
                         add=True)
        pltpu.make_async_copy(rows_a, acc_sh.at[dst_v.at[ja]],
                              sem_sa).wait()
        pltpu.async_copy(hp_hbm.at[_sidx(ja + 2)], rows_a, sem_a)
        pltpu.make_async_copy(rows_b, acc_sh.at[dst_v.at[ja + 1]],
                              sem_sb).wait()

        @pl.when(ja + 3 < NCHUNK)
        def _():
            pltpu.async_copy(hp_hbm.at[_sidx(ja + 3)], rows_b, sem_b)

        return _

    lax.fori_loop(0, NCHUNK // 2, _edge_body, None)
    last = NCHUNK - 1
    pltpu.make_async_copy(hp_hbm.at[_sidx(last)], rows_a, sem_a).wait()
    pltpu.sync_copy(rows_a, acc_sh.at[dst_v.at[last]], add=True)
    plsc.subcore_barrier()

    for k in range(ZCOPIES):
        base = sid * RPT + k * RB
        pltpu.sync_copy(acc_sh.at[pl.ds(base, RB)], rows_a.at[pl.ds(0, RB)])
        pltpu.sync_copy(rows_a.at[pl.ds(0, RB)],
                        out_hbm.at[cid, pl.ds(base, RB)])


def _dense_body(x_ref, w_ref, degp_ref, hp_ref, dis_ref):
    s = degp_ref[...]
    deg = s[0:1, :N] + s[1:2, :N] + 1.0
    dis = jnp.reshape(lax.rsqrt(deg), (N, 1))
    dis_ref[...] = dis
    h = jnp.dot(x_ref[...], w_ref[...], preferred_element_type=jnp.float32)
    hp_ref[...] = h * dis


def _final_body(sp_ref, hp_ref, dis_ref, b_ref, g_ref, be_ref, y_ref):
    agg = (sp_ref[0, :N, :D_OUT] + sp_ref[1, :N, :D_OUT]
           + hp_ref[:, :D_OUT])
    o = agg * dis_ref[...] + b_ref[...]
    o = jnp.maximum(o, 0.0)
    mean = jnp.mean(o, axis=0, keepdims=True)
    c = o - mean
    var = jnp.mean(c * c, axis=0, keepdims=True)
    y_ref[...] = c * lax.rsqrt(var + 1e-5) * g_ref[...] + be_ref[...]


def kernel(x, edge_index, W, b, gamma, beta):
    src = edge_index[0].reshape(NW, EPW)
    dst = edge_index[1].reshape(NW, NCHUNK, CHUNK)

    deg_p = _deg_kernel(dst)
    W_pad = jnp.pad(W, ((0, 0), (0, DP - D_OUT)))

    hp, dis = pl.pallas_call(
        _dense_body,
        out_shape=[
            jax.ShapeDtypeStruct((N, DP), jnp.float32),
            jax.ShapeDtypeStruct((N, 1), jnp.float32),
        ],
    )(x, W_pad, deg_p)

    s_p = _agg_kernel(hp, src, dst)

    y = pl.pallas_call(
        _final_body,
        out_shape=jax.ShapeDtypeStruct((N, D_OUT), jnp.float32),
    )(s_p, hp, dis,
      b.reshape(1, D_OUT), gamma.reshape(1, D_OUT), beta.reshape(1, D_OUT))
    return y


# speedup vs baseline: 1.1861x; 1.1861x over previous
import functools

import jax
import jax.numpy as jnp
from jax import lax
from jax.experimental import pallas as pl
from jax.experimental.pallas import tpu as pltpu
from jax.experimental.pallas import tpu_sc as plsc

N = 10000
E = 320000
D_IN = 200
D_OUT = 64

NC = 2
NS = 16
NW = NC * NS
EPW = E // NW
CHUNK = 80
NCHUNK = EPW // CHUNK
NPAD = 10240
RPT = NPAD // NS
RB = 80
ZCOPIES = RPT // RB
DP = 128
DEGD = 5

_mesh = plsc.VectorSubcoreMesh(core_axis_name="c", subcore_axis_name="s")


@functools.partial(
    pl.kernel,
    mesh=_mesh,
    out_type=jax.ShapeDtypeStruct((NC, NPAD), jnp.float32),
    scratch_types=[
        pltpu.VMEM((NCHUNK, CHUNK), jnp.int32),
        pltpu.VMEM((112,), jnp.float32),
        pltpu.VMEM((RPT,), jnp.float32),
        pltpu.VMEM_SHARED((NPAD,), jnp.float32),
        pltpu.SemaphoreType.DMA,
    ],
)
def _deg_kernel(dst_hbm, out_hbm, idx_v, ones_v, stage_v, acc_sh, sem):
    cid = lax.axis_index("c")
    sid = lax.axis_index("s")
    wid = cid * NS + sid

    pltpu.sync_copy(dst_hbm.at[wid], idx_v)

    for i in range(7):
        ones_v[pl.ds(i * 16, 16)] = jnp.ones((16,), jnp.float32)

    def _zero_body(i, _):
        stage_v[pl.ds(i * 16, 16)] = jnp.zeros((16,), jnp.float32)
        return _

    lax.fori_loop(0, RPT // 16, _zero_body, None)

    pltpu.sync_copy(stage_v, acc_sh.at[pl.ds(sid * RPT, RPT)])
    plsc.subcore_barrier()

    def _scatter_body(j, _):
        for k in range(DEGD):
            pltpu.async_copy(ones_v.at[pl.ds(0, CHUNK)],
                             acc_sh.at[idx_v.at[DEGD * j + k]], sem,
                             add=True)
        for k in range(DEGD):
            pltpu.make_async_copy(ones_v.at[pl.ds(0, CHUNK)],
                                  acc_sh.at[idx_v.at[DEGD * j + k]],
                                  sem).wait()
        return _

    lax.fori_loop(0, NCHUNK // DEGD, _scatter_body, None)
    plsc.subcore_barrier()

    pltpu.sync_copy(acc_sh.at[pl.ds(sid * RPT, RPT)], stage_v)
    pltpu.sync_copy(stage_v, out_hbm.at[cid, pl.ds(sid * RPT, RPT)])


@functools.partial(
    pl.kernel,
    mesh=_mesh,
    out_type=jax.ShapeDtypeStruct((NC, NPAD, DP), jnp.float32),
    scratch_types=[
        pltpu.VMEM((EPW,), jnp.int32),
        pltpu.VMEM((NCHUNK, CHUNK), jnp.int32),
        pltpu.VMEM((CHUNK, DP), jnp.float32),
        pltpu.VMEM((CHUNK, DP), jnp.float32),
        pltpu.VMEM_SHARED((NPAD, DP), jnp.float32),
        pltpu.SemaphoreType.DMA,
        pltpu.SemaphoreType.DMA,
    ],
)
def _agg_kernel(hp_hbm, src_hbm, dst_hbm, out_hbm,
                src_v, dst_v, rows_a, rows_b, acc_sh, sem_a, sem_b):
    cid = lax.axis_index("c")
    sid = lax.axis_index("s")
    wid = cid * NS + sid

    pltpu.sync_copy(src_hbm.at[wid], src_v)
    pltpu.sync_copy(dst_hbm.at[wid], dst_v)

    def _zero_body(i, _):
        for k in range(DP // 16):
            rows_a[i, pl.ds(k * 16, 16)] = jnp.zeros((16,), jnp.float32)
        return _

    lax.fori_loop(0, RB, _zero_body, None)
    for k in range(ZCOPIES):
        pltpu.sync_copy(rows_a.at[pl.ds(0, RB)],
                        acc_sh.at[pl.ds(sid * RPT + k * RB, RB)])
    plsc.subcore_barrier()

    def _sidx(j):
        return src_v.at[pl.ds(j * CHUNK, CHUNK)]

    pltpu.async_copy(hp_hbm.at[_sidx(0)], rows_a, sem_a)

    def _edge_body(t, _):
        ja = 2 * t
        pltpu.async_copy(hp_hbm.at[_sidx(ja + 1)], rows_b, sem_b)
        pltpu.make_async_copy(hp_hbm.at[_sidx(ja)], rows_a, sem_a).wait()
        pltpu.sync_copy(rows_a, acc_sh.at[dst_v.at[ja]], add=True)
        pltpu.async_copy(hp_hbm.at[_sidx(ja + 2)], rows_a, sem_a)
        pltpu.make_async_copy(hp_hbm.at[_sidx(ja + 1)], rows_b, sem_b).wait()
        pltpu.sync_copy(rows_b, acc_sh.at[dst_v.at[ja + 1]], add=True)
        return _

    lax.fori_loop(0, NCHUNK // 2, _edge_body, None)
    last = NCHUNK - 1
    pltpu.make_async_copy(hp_hbm.at[_sidx(last)], rows_a, sem_a).wait()
    pltpu.sync_copy(rows_a, acc_sh.at[dst_v.at[last]], add=True)
    plsc.subcore_barrier()

    for k in range(ZCOPIES):
        base = sid * RPT + k * RB
        pltpu.sync_copy(acc_sh.at[pl.ds(base, RB)], rows_a.at[pl.ds(0, RB)])
        pltpu.sync_copy(rows_a.at[pl.ds(0, RB)],
                        out_hbm.at[cid, pl.ds(base, RB)])


def _dense_body(x_ref, w_ref, degp_ref, hp_ref, dis_ref):
    s = degp_ref[...]
    deg = s[0:1, :N] + s[1:2, :N] + 1.0
    dis = jnp.reshape(lax.rsqrt(deg), (N, 1))
    dis_ref[...] = dis
    h = jnp.dot(x_ref[...], w_ref[...], preferred_element_type=jnp.float32)
    hp_ref[...] = h * dis


def _final_body(sp_ref, hp_ref, dis_ref, b_ref, g_ref, be_ref, y_ref):
    agg = (sp_ref[0, :N, :D_OUT] + sp_ref[1, :N, :D_OUT]
           + hp_ref[:, :D_OUT])
    o = agg * dis_ref[...] + b_ref[...]
    o = jnp.maximum(o, 0.0)
    mean = jnp.mean(o, axis=0, keepdims=True)
    c = o - mean
    var = jnp.mean(c * c, axis=0, keepdims=True)
    y_ref[...] = c * lax.rsqrt(var + 1e-5) * g_ref[...] + be_ref[...]


def kernel(x, edge_index, W, b, gamma, beta):
    src = edge_index[0].reshape(NW, EPW)
    dst = edge_index[1].reshape(NW, NCHUNK, CHUNK)

    deg_p = _deg_kernel(dst)
    W_pad = jnp.pad(W, ((0, 0), (0, DP - D_OUT)))

    hp, dis = pl.pallas_call(
        _dense_body,
        out_shape=[
            jax.ShapeDtypeStruct((N, DP), jnp.float32),
            jax.ShapeDtypeStruct((N, 1), jnp.float32),
        ],
    )(x, W_pad, deg_p)

    s_p = _agg_kernel(hp, src, dst)

    y = pl.pallas_call(
        _final_body,
        out_shape=jax.ShapeDtypeStruct((N, D_OUT), jnp.float32),
    )(s_p, hp, dis,
      b.reshape(1, D_OUT), gamma.reshape(1, D_OUT), beta.reshape(1, D_OUT))
    return y


# speedup vs baseline: 1.3624x; 1.1486x over previous
import functools

import jax
import jax.numpy as jnp
from jax import lax
from jax.experimental import pallas as pl
from jax.experimental.pallas import tpu as pltpu
from jax.experimental.pallas import tpu_sc as plsc

N = 10000
E = 320000
D_IN = 200
D_OUT = 64

NC = 2
NS = 16
NW = NC * NS
EPW = E // NW
CHUNK = 80
NCHUNK = EPW // CHUNK
NPAD = 10240
RPT = NPAD // NS
RB = 80
ZCOPIES = RPT // RB
DP = 64
DEGD = 5

_mesh = plsc.VectorSubcoreMesh(core_axis_name="c", subcore_axis_name="s")


@functools.partial(
    pl.kernel,
    mesh=_mesh,
    out_type=jax.ShapeDtypeStruct((NC, NPAD), jnp.float32),
    scratch_types=[
        pltpu.VMEM((NCHUNK, CHUNK), jnp.int32),
        pltpu.VMEM((112,), jnp.float32),
        pltpu.VMEM((RPT,), jnp.float32),
        pltpu.VMEM_SHARED((NPAD,), jnp.float32),
        pltpu.SemaphoreType.DMA,
    ],
)
def _deg_kernel(dst_hbm, out_hbm, idx_v, ones_v, stage_v, acc_sh, sem):
    cid = lax.axis_index("c")
    sid = lax.axis_index("s")
    wid = cid * NS + sid

    pltpu.sync_copy(dst_hbm.at[wid], idx_v)

    for i in range(7):
        ones_v[pl.ds(i * 16, 16)] = jnp.ones((16,), jnp.float32)

    def _zero_body(i, _):
        stage_v[pl.ds(i * 16, 16)] = jnp.zeros((16,), jnp.float32)
        return _

    lax.fori_loop(0, RPT // 16, _zero_body, None)

    pltpu.sync_copy(stage_v, acc_sh.at[pl.ds(sid * RPT, RPT)])
    plsc.subcore_barrier()

    def _scatter_body(j, _):
        for k in range(DEGD):
            pltpu.async_copy(ones_v.at[pl.ds(0, CHUNK)],
                             acc_sh.at[idx_v.at[DEGD * j + k]], sem,
                             add=True)
        for k in range(DEGD):
            pltpu.make_async_copy(ones_v.at[pl.ds(0, CHUNK)],
                                  acc_sh.at[idx_v.at[DEGD * j + k]],
                                  sem).wait()
        return _

    lax.fori_loop(0, NCHUNK // DEGD, _scatter_body, None)
    plsc.subcore_barrier()

    pltpu.sync_copy(acc_sh.at[pl.ds(sid * RPT, RPT)], stage_v)
    pltpu.sync_copy(stage_v, out_hbm.at[cid, pl.ds(sid * RPT, RPT)])


@functools.partial(
    pl.kernel,
    mesh=_mesh,
    compiler_params=pltpu.CompilerParams(use_tc_tiling_on_sc=False),
    out_type=jax.ShapeDtypeStruct((NC, NPAD, DP), jnp.float32),
    scratch_types=[
        pltpu.VMEM((EPW,), jnp.int32),
        pltpu.VMEM((NCHUNK, CHUNK), jnp.int32),
        pltpu.VMEM((CHUNK, DP), jnp.float32),
        pltpu.VMEM((CHUNK, DP), jnp.float32),
        pltpu.VMEM_SHARED((NPAD, DP), jnp.float32),
        pltpu.SemaphoreType.DMA,
        pltpu.SemaphoreType.DMA,
    ],
)
def _agg_kernel(hp_hbm, src_hbm, dst_hbm, out_hbm,
                src_v, dst_v, rows_a, rows_b, acc_sh, sem_a, sem_b):
    cid = lax.axis_index("c")
    sid = lax.axis_index("s")
    wid = cid * NS + sid

    pltpu.sync_copy(src_hbm.at[wid], src_v)
    pltpu.sync_copy(dst_hbm.at[wid], dst_v)

    def _zero_body(i, _):
        for k in range(DP // 16):
            rows_a[i, pl.ds(k * 16, 16)] = jnp.zeros((16,), jnp.float32)
        return _

    lax.fori_loop(0, RB, _zero_body, None)
    for k in range(ZCOPIES):
        pltpu.sync_copy(rows_a.at[pl.ds(0, RB)],
                        acc_sh.at[pl.ds(sid * RPT + k * RB, RB)])
    plsc.subcore_barrier()

    def _sidx(j):
        return src_v.at[pl.ds(j * CHUNK, CHUNK)]

    pltpu.async_copy(hp_hbm.at[_sidx(0)], rows_a, sem_a)

    def _edge_body(t, _):
        ja = 2 * t
        pltpu.async_copy(hp_hbm.at[_sidx(ja + 1)], rows_b, sem_b)
        pltpu.make_async_copy(hp_hbm.at[_sidx(ja)], rows_a, sem_a).wait()
        pltpu.sync_copy(rows_a, acc_sh.at[dst_v.at[ja]], add=True)
        pltpu.async_copy(hp_hbm.at[_sidx(ja + 2)], rows_a, sem_a)
        pltpu.make_async_copy(hp_hbm.at[_sidx(ja + 1)], rows_b, sem_b).wait()
        pltpu.sync_copy(rows_b, acc_sh.at[dst_v.at[ja + 1]], add=True)
        return _

    lax.fori_loop(0, NCHUNK // 2, _edge_body, None)
    last = NCHUNK - 1
    pltpu.make_async_copy(hp_hbm.at[_sidx(last)], rows_a, sem_a).wait()
    pltpu.sync_copy(rows_a, acc_sh.at[dst_v.at[last]], add=True)
    plsc.subcore_barrier()

    for k in range(ZCOPIES):
        base = sid * RPT + k * RB
        pltpu.sync_copy(acc_sh.at[pl.ds(base, RB)], rows_a.at[pl.ds(0, RB)])
        pltpu.sync_copy(rows_a.at[pl.ds(0, RB)],
                        out_hbm.at[cid, pl.ds(base, RB)])


def _dense_body(x_ref, w_ref, degp_ref, hp_ref, dis_ref):
    s = degp_ref[...]
    deg = s[0:1, :N] + s[1:2, :N] + 1.0
    dis = jnp.reshape(lax.rsqrt(deg), (N, 1))
    dis_ref[...] = dis
    h = jnp.dot(x_ref[...], w_ref[...], preferred_element_type=jnp.float32)
    hp_ref[...] = h * dis


def _final_body(sp_ref, hp_ref, dis_ref, b_ref, g_ref, be_ref, y_ref):
    agg = (sp_ref[0, :N, :D_OUT] + sp_ref[1, :N, :D_OUT]
           + hp_ref[:, :D_OUT])
    o = agg * dis_ref[...] + b_ref[...]
    o = jnp.maximum(o, 0.0)
    mean = jnp.mean(o, axis=0, keepdims=True)
    c = o - mean
    var = jnp.mean(c * c, axis=0, keepdims=True)
    y_ref[...] = c * lax.rsqrt(var + 1e-5) * g_ref[...] + be_ref[...]


def kernel(x, edge_index, W, b, gamma, beta):
    src = edge_index[0].reshape(NW, EPW)
    dst = edge_index[1].reshape(NW, NCHUNK, CHUNK)

    deg_p = _deg_kernel(dst)
    W_pad = jnp.pad(W, ((0, 0), (0, DP - D_OUT)))

    hp, dis = pl.pallas_call(
        _dense_body,
        out_shape=[
            jax.ShapeDtypeStruct((N, DP), jnp.float32),
            jax.ShapeDtypeStruct((N, 1), jnp.float32),
        ],
    )(x, W_pad, deg_p)

    s_p = _agg_kernel(hp, src, dst)

    y = pl.pallas_call(
        _final_body,
        out_shape=jax.ShapeDtypeStruct((N, D_OUT), jnp.float32),
    )(s_p, hp, dis,
      b.reshape(1, D_OUT), gamma.reshape(1, D_OUT), beta.reshape(1, D_OUT))
    return y


# speedup vs baseline: 1.4674x; 1.0771x over previous
import functools

import jax
import jax.numpy as jnp
from jax import lax
from jax.experimental import pallas as pl
from jax.experimental.pallas import tpu as pltpu
from jax.experimental.pallas import tpu_sc as plsc

N = 10000
E = 320000
D_IN = 200
D_OUT = 64

NC = 2
NS = 16
NW = NC * NS
EPW = E // NW
CHUNK = 80
NCHUNK = EPW // CHUNK
NPAD = 10240
RPT = NPAD // NS
RB = 80
ZCOPIES = RPT // RB
DP = 64
HPW = 128
DEGD = 5

_mesh = plsc.VectorSubcoreMesh(core_axis_name="c", subcore_axis_name="s")


@functools.partial(
    pl.kernel,
    mesh=_mesh,
    out_type=jax.ShapeDtypeStruct((NC, NPAD), jnp.float32),
    scratch_types=[
        pltpu.VMEM((NCHUNK, CHUNK), jnp.int32),
        pltpu.VMEM((112,), jnp.float32),
        pltpu.VMEM((RPT,), jnp.float32),
        pltpu.VMEM_SHARED((NPAD,), jnp.float32),
        pltpu.SemaphoreType.DMA,
    ],
)
def _deg_kernel(dst_hbm, out_hbm, idx_v, ones_v, stage_v, acc_sh, sem):
    cid = lax.axis_index("c")
    sid = lax.axis_index("s")
    wid = cid * NS + sid

    pltpu.sync_copy(dst_hbm.at[wid], idx_v)

    for i in range(7):
        ones_v[pl.ds(i * 16, 16)] = jnp.ones((16,), jnp.float32)

    def _zero_body(i, _):
        stage_v[pl.ds(i * 16, 16)] = jnp.zeros((16,), jnp.float32)
        return _

    lax.fori_loop(0, RPT // 16, _zero_body, None)

    pltpu.sync_copy(stage_v, acc_sh.at[pl.ds(sid * RPT, RPT)])
    plsc.subcore_barrier()

    def _scatter_body(j, _):
        for k in range(DEGD):
            pltpu.async_copy(ones_v.at[pl.ds(0, CHUNK)],
                             acc_sh.at[idx_v.at[DEGD * j + k]], sem,
                             add=True)
        for k in range(DEGD):
            pltpu.make_async_copy(ones_v.at[pl.ds(0, CHUNK)],
                                  acc_sh.at[idx_v.at[DEGD * j + k]],
                                  sem).wait()
        return _

    lax.fori_loop(0, NCHUNK // DEGD, _scatter_body, None)
    plsc.subcore_barrier()

    pltpu.sync_copy(acc_sh.at[pl.ds(sid * RPT, RPT)], stage_v)
    pltpu.sync_copy(stage_v, out_hbm.at[cid, pl.ds(sid * RPT, RPT)])


@functools.partial(
    pl.kernel,
    mesh=_mesh,
    compiler_params=pltpu.CompilerParams(use_tc_tiling_on_sc=False),
    out_type=jax.ShapeDtypeStruct((NC, NPAD, HPW), jnp.float32),
    scratch_types=[
        pltpu.VMEM((EPW,), jnp.int32),
        pltpu.VMEM((NCHUNK, CHUNK), jnp.int32),
        pltpu.VMEM((CHUNK, DP), jnp.float32),
        pltpu.VMEM((CHUNK, DP), jnp.float32),
        pltpu.VMEM_SHARED((NPAD, DP), jnp.float32),
        pltpu.SemaphoreType.DMA,
        pltpu.SemaphoreType.DMA,
    ],
)
def _agg_kernel(hp_hbm, src_hbm, dst_hbm, out_hbm,
                src_v, dst_v, rows_a, rows_b, acc_sh, sem_a, sem_b):
    cid = lax.axis_index("c")
    sid = lax.axis_index("s")
    wid = cid * NS + sid

    pltpu.sync_copy(src_hbm.at[wid], src_v)
    pltpu.sync_copy(dst_hbm.at[wid], dst_v)

    def _dbl_body(i, _):
        v = src_v[pl.ds(i * 16, 16)]
        src_v[pl.ds(i * 16, 16)] = v + v
        return _

    lax.fori_loop(0, EPW // 16, _dbl_body, None)

    def _zero_body(i, _):
        for k in range(DP // 16):
            rows_a[i, pl.ds(k * 16, 16)] = jnp.zeros((16,), jnp.float32)
        return _

    lax.fori_loop(0, RB, _zero_body, None)
    for k in range(ZCOPIES):
        pltpu.sync_copy(rows_a.at[pl.ds(0, RB)],
                        acc_sh.at[pl.ds(sid * RPT + k * RB, RB)])
    plsc.subcore_barrier()

    def _sidx(j):
        return src_v.at[pl.ds(j * CHUNK, CHUNK)]

    pltpu.async_copy(hp_hbm.at[_sidx(0)], rows_a, sem_a)

    def _edge_body(t, _):
        ja = 2 * t
        pltpu.async_copy(hp_hbm.at[_sidx(ja + 1)], rows_b, sem_b)
        pltpu.make_async_copy(hp_hbm.at[_sidx(ja)], rows_a, sem_a).wait()
        pltpu.sync_copy(rows_a, acc_sh.at[dst_v.at[ja]], add=True)
        pltpu.async_copy(hp_hbm.at[_sidx(ja + 2)], rows_a, sem_a)
        pltpu.make_async_copy(hp_hbm.at[_sidx(ja + 1)], rows_b, sem_b).wait()
        pltpu.sync_copy(rows_b, acc_sh.at[dst_v.at[ja + 1]], add=True)
        return _

    lax.fori_loop(0, NCHUNK // 2, _edge_body, None)
    last = NCHUNK - 1
    pltpu.make_async_copy(hp_hbm.at[_sidx(last)], rows_a, sem_a).wait()
    pltpu.sync_copy(rows_a, acc_sh.at[dst_v.at[last]], add=True)
    plsc.subcore_barrier()

    for k in range(ZCOPIES):
        base = sid * RPT + k * RB
        pltpu.sync_copy(acc_sh.at[pl.ds(base, RB)], rows_a.at[pl.ds(0, RB)])
        pltpu.sync_copy(rows_a.at[pl.ds(0, RB)],
                        out_hbm.at[cid, pl.ds(base, RB), pl.ds(0, DP)])


def _dense_body(x_ref, w_ref, degp_ref, hp_ref, dis_ref):
    s = degp_ref[...]
    deg = s[0:1, :N] + s[1:2, :N] + 1.0
    dis = jnp.reshape(lax.rsqrt(deg), (N, 1))
    dis_ref[...] = dis
    h = jnp.dot(x_ref[...], w_ref[...], preferred_element_type=jnp.float32)
    hp_ref[...] = h * dis


def _final_body(sp_ref, hp_ref, dis_ref, b_ref, g_ref, be_ref, y_ref):
    agg = (sp_ref[0, :, :D_OUT] + sp_ref[1, :, :D_OUT]
           + hp_ref[:, :D_OUT])
    o = agg * dis_ref[...] + b_ref[...]
    o = jnp.maximum(o, 0.0)
    mean = jnp.mean(o, axis=0, keepdims=True)
    c = o - mean
    var = jnp.mean(c * c, axis=0, keepdims=True)
    y_ref[...] = c * lax.rsqrt(var + 1e-5) * g_ref[...] + be_ref[...]


def kernel(x, edge_index, W, b, gamma, beta):
    src = edge_index[0].reshape(NW, EPW)
    dst = edge_index[1].reshape(NW, NCHUNK, CHUNK)

    deg_p = _deg_kernel(dst)
    W_pad = jnp.pad(W, ((0, 0), (0, HPW - D_OUT)))

    hp, dis = pl.pallas_call(
        _dense_body,
        out_shape=[
            jax.ShapeDtypeStruct((N, HPW), jnp.float32),
            jax.ShapeDtypeStruct((N, 1), jnp.float32),
        ],
    )(x, W_pad, deg_p)

    hp2 = hp.reshape(2 * N, DP)
    s_p = _agg_kernel(hp2, src, dst)

    y = pl.pallas_call(
        _final_body,
        grid=(1,),
        in_specs=[
            pl.BlockSpec((NC, N, HPW), lambda i: (0, 0, 0)),
            pl.BlockSpec((N, HPW), lambda i: (0, 0)),
            pl.BlockSpec((N, 1), lambda i: (0, 0)),
            pl.BlockSpec((1, D_OUT), lambda i: (0, 0)),
            pl.BlockSpec((1, D_OUT), lambda i: (0, 0)),
            pl.BlockSpec((1, D_OUT), lambda i: (0, 0)),
        ],
        out_specs=pl.BlockSpec((N, D_OUT), lambda i: (0, 0)),
        out_shape=jax.ShapeDtypeStruct((N, D_OUT), jnp.float32),
    )(s_p, hp, dis,
      b.reshape(1, D_OUT), gamma.reshape(1, D_OUT), beta.reshape(1, D_OUT))
    return y


# speedup vs baseline: 1.5810x; 1.0774x over previous
import functools

import jax
import jax.numpy as jnp
from jax import lax
from jax.experimental import pallas as pl
from jax.experimental.pallas import tpu as pltpu
from jax.experimental.pallas import tpu_sc as plsc

N = 10000
E = 320000
D_IN = 200
D_OUT = 64

NC = 2
NS = 16
NW = NC * NS
EPW = E // NW
CHUNK = 80
NCHUNK = EPW // CHUNK
NPAD = 10240
RPT = NPAD // NS
RB = 80
ZCOPIES = RPT // RB
DP = 64
HPW = 128
DEGD = 5

_mesh = plsc.VectorSubcoreMesh(core_axis_name="c", subcore_axis_name="s")


@functools.partial(
    pl.kernel,
    mesh=_mesh,
    compiler_params=pltpu.CompilerParams(use_tc_tiling_on_sc=False),
    out_type=jax.ShapeDtypeStruct((NC, NPAD), jnp.float32),
    scratch_types=[
        pltpu.VMEM((EPW,), jnp.int32),
        pltpu.VMEM((112,), jnp.float32),
        pltpu.VMEM((RPT,), jnp.float32),
        pltpu.VMEM_SHARED((NPAD,), jnp.float32),
        pltpu.SemaphoreType.DMA,
    ],
)
def _deg_kernel(e_hbm, out_hbm, idx_v, ones_v, stage_v, acc_sh, sem):
    cid = lax.axis_index("c")
    sid = lax.axis_index("s")
    wid = cid * NS + sid

    pltpu.sync_copy(e_hbm.at[1, pl.ds(wid * EPW, EPW)], idx_v)

    for i in range(7):
        ones_v[pl.ds(i * 16, 16)] = jnp.ones((16,), jnp.float32)

    def _zero_body(i, _):
        stage_v[pl.ds(i * 16, 16)] = jnp.zeros((16,), jnp.float32)
        return _

    lax.fori_loop(0, RPT // 16, _zero_body, None)

    pltpu.sync_copy(stage_v, acc_sh.at[pl.ds(sid * RPT, RPT)])
    plsc.subcore_barrier()

    def _scatter_body(j, _):
        for k in range(DEGD):
            c = (DEGD * j + k) * CHUNK
            pltpu.async_copy(ones_v.at[pl.ds(0, CHUNK)],
                             acc_sh.at[idx_v.at[pl.ds(c, CHUNK)]], sem,
                             add=True)
        for k in range(DEGD):
            c = (DEGD * j + k) * CHUNK
            pltpu.make_async_copy(ones_v.at[pl.ds(0, CHUNK)],
                                  acc_sh.at[idx_v.at[pl.ds(c, CHUNK)]],
                                  sem).wait()
        return _

    lax.fori_loop(0, NCHUNK // DEGD, _scatter_body, None)
    plsc.subcore_barrier()

    pltpu.sync_copy(acc_sh.at[pl.ds(sid * RPT, RPT)], stage_v)
    pltpu.sync_copy(stage_v, out_hbm.at[cid, pl.ds(sid * RPT, RPT)])


@functools.partial(
    pl.kernel,
    mesh=_mesh,
    compiler_params=pltpu.CompilerParams(use_tc_tiling_on_sc=False),
    out_type=jax.ShapeDtypeStruct((NC, NPAD, HPW), jnp.float32),
    scratch_types=[
        pltpu.VMEM((EPW,), jnp.int32),
        pltpu.VMEM((EPW,), jnp.int32),
        pltpu.VMEM((CHUNK, DP), jnp.float32),
        pltpu.VMEM((CHUNK, DP), jnp.float32),
        pltpu.VMEM_SHARED((NPAD, DP), jnp.float32),
        pltpu.VMEM_SHARED((NPAD, DP), jnp.float32),
        pltpu.SemaphoreType.DMA,
        pltpu.SemaphoreType.DMA,
    ],
)
def _agg_kernel(hp_hbm, e_hbm, out_hbm,
                src_v, dst_v, rows_a, rows_b, hp_sh, acc_sh, sem_a, sem_b):
    cid = lax.axis_index("c")
    sid = lax.axis_index("s")
    wid = cid * NS + sid
    ebase = wid * EPW

    pltpu.sync_copy(e_hbm.at[0, pl.ds(ebase, EPW)], src_v)
    pltpu.sync_copy(e_hbm.at[1, pl.ds(ebase, EPW)], dst_v)

    pltpu.sync_copy(hp_hbm.at[pl.ds(sid * RPT, RPT)],
                    hp_sh.at[pl.ds(sid * RPT, RPT)])

    def _zero_body(i, _):
        for k in range(DP // 16):
            rows_a[i, pl.ds(k * 16, 16)] = jnp.zeros((16,), jnp.float32)
        return _

    lax.fori_loop(0, RB, _zero_body, None)
    for k in range(ZCOPIES):
        pltpu.sync_copy(rows_a.at[pl.ds(0, RB)],
                        acc_sh.at[pl.ds(sid * RPT + k * RB, RB)])
    plsc.subcore_barrier()

    def _sidx(j):
        return src_v.at[pl.ds(j * CHUNK, CHUNK)]

    def _didx(j):
        return dst_v.at[pl.ds(j * CHUNK, CHUNK)]

    pltpu.async_copy(hp_sh.at[_sidx(0)], rows_a, sem_a)

    def _edge_body(t, _):
        ja = 2 * t
        pltpu.async_copy(hp_sh.at[_sidx(ja + 1)], rows_b, sem_b)
        pltpu.make_async_copy(hp_sh.at[_sidx(ja)], rows_a, sem_a).wait()
        pltpu.sync_copy(rows_a, acc_sh.at[_didx(ja)], add=True)
        pltpu.async_copy(hp_sh.at[_sidx(ja + 2)], rows_a, sem_a)
        pltpu.make_async_copy(hp_sh.at[_sidx(ja + 1)], rows_b, sem_b).wait()
        pltpu.sync_copy(rows_b, acc_sh.at[_didx(ja + 1)], add=True)
        return _

    lax.fori_loop(0, NCHUNK // 2, _edge_body, None)
    last = NCHUNK - 1
    pltpu.make_async_copy(hp_sh.at[_sidx(last)], rows_a, sem_a).wait()
    pltpu.sync_copy(rows_a, acc_sh.at[_didx(last)], add=True)
    plsc.subcore_barrier()

    for k in range(ZCOPIES):
        base = sid * RPT + k * RB
        pltpu.sync_copy(acc_sh.at[pl.ds(base, RB)], rows_a.at[pl.ds(0, RB)])
        pltpu.sync_copy(rows_a.at[pl.ds(0, RB)],
                        out_hbm.at[cid, pl.ds(base, RB), pl.ds(0, DP)])


def _dense_body(x_ref, w_ref, degp_ref, hp_ref, dis_ref):
    s = degp_ref[...]
    deg = s[0:1, :N] + s[1:2, :N] + 1.0
    dis = jnp.reshape(lax.rsqrt(deg), (N, 1))
    dis_ref[...] = dis
    hp_ref[N:, :] = jnp.zeros((NPAD - N, DP), jnp.float32)
    h = jnp.dot(x_ref[...], w_ref[...], preferred_element_type=jnp.float32)
    hp_ref[:N, :] = h * dis


def _final_body(sp_ref, hp_ref, dis_ref, b_ref, g_ref, be_ref, y_ref):
    agg = (sp_ref[0, :, :D_OUT] + sp_ref[1, :, :D_OUT]
           + hp_ref[:, :D_OUT])
    o = agg * dis_ref[...] + b_ref[...]
    o = jnp.maximum(o, 0.0)
    mean = jnp.mean(o, axis=0, keepdims=True)
    c = o - mean
    var = jnp.mean(c * c, axis=0, keepdims=True)
    y_ref[...] = c * lax.rsqrt(var + 1e-5) * g_ref[...] + be_ref[...]


def kernel(x, edge_index, W, b, gamma, beta):
    deg_p = _deg_kernel(edge_index)

    hp, dis = pl.pallas_call(
        _dense_body,
        out_shape=[
            jax.ShapeDtypeStruct((NPAD, DP), jnp.float32),
            jax.ShapeDtypeStruct((N, 1), jnp.float32),
        ],
    )(x, W, deg_p)

    s_p = _agg_kernel(hp, edge_index)

    y = pl.pallas_call(
        _final_body,
        grid=(1,),
        in_specs=[
            pl.BlockSpec((NC, N, HPW), lambda i: (0, 0, 0)),
            pl.BlockSpec((N, DP), lambda i: (0, 0)),
            pl.BlockSpec((N, 1), lambda i: (0, 0)),
            pl.BlockSpec((1, D_OUT), lambda i: (0, 0)),
            pl.BlockSpec((1, D_OUT), lambda i: (0, 0)),
            pl.BlockSpec((1, D_OUT), lambda i: (0, 0)),
        ],
        out_specs=pl.BlockSpec((N, D_OUT), lambda i: (0, 0)),
        out_shape=jax.ShapeDtypeStruct((N, D_OUT), jnp.float32),
    )(s_p, hp, dis,
      b.reshape(1, D_OUT), gamma.reshape(1, D_OUT), beta.reshape(1, D_OUT))
    return y
